# baseline (speedup 1.0000x reference)
import jax
import jax.numpy as jnp
from jax import lax
from jax.experimental import pallas as pl
from jax.experimental.pallas import tpu as pltpu
from jax.experimental.pallas import tpu_sc as plsc

N = 100000
E = 1600000
IN = 28
HID = 128
G = 128
EDIM = 20

NC, NS, L = 2, 16, 16
KB = 8
KE = KB * 128
NBLK = (E // 128) // KB
TROW = (E // 128) - NBLK * KB
NSTRIPE = N // NS
NP = 102400

BR = 2000
EG = (E // 8) // BR


def _edge_body(ex_ref, w_ref, b_ref, out_ref):
    v = (jnp.dot(ex_ref[...], w_ref[...], preferred_element_type=jnp.float32)
         + b_ref[...])
    out_ref[0, :, :] = v[:, 0:128]
    out_ref[1, :, :] = v[:, 128:256]


def _edge_mlp(ex_r8, wb, bb):
    return pl.pallas_call(
        _edge_body,
        grid=(EG,),
        in_specs=[
            pl.BlockSpec((BR, 8 * EDIM), lambda i: (i, 0)),
            pl.BlockSpec((8 * EDIM, 256), lambda i: (0, 0)),
            pl.BlockSpec((1, 256), lambda i: (0, 0)),
        ],
        out_specs=pl.BlockSpec((2, BR, 128), lambda i: (0, i, 0)),
        out_shape=jax.ShapeDtypeStruct((2, E // 8, 128), jnp.float32),
    )(ex_r8, wb, bb)


def _sc_body(tab, src2, dstm, e2, out,
             idx_s2, idx_d2, xg, eb, aggr_sh, sem, sem_i, sem_w):
    c = lax.axis_index("c")
    s = lax.axis_index("s")

    def _z(i, _):
        xg[i, :] = jnp.zeros((L,), jnp.float32)
        return 0
    lax.fori_loop(0, KE, _z, 0)
    for k in range(NSTRIPE // KE):
        pltpu.sync_copy(xg.at[pl.ds(0, KE)],
                        aggr_sh.at[pl.ds(s * NSTRIPE + k * KE, KE)])
    rem = NSTRIPE - (NSTRIPE // KE) * KE
    pltpu.sync_copy(xg.at[pl.ds(0, rem)],
                    aggr_sh.at[pl.ds(s * NSTRIPE + NSTRIPE - rem, rem)])

    plsc.subcore_barrier()

    nit = (NBLK - s + NS - 1) // NS
    maxrow = (E // 128) - KB

    pltpu.sync_copy(src2.at[c, pl.ds(s * KB, KB)], idx_s2.at[0])
    pltpu.sync_copy(dstm.at[pl.ds(s * KB, KB)], idx_d2.at[0])

    def _blk(i, _):
        par = lax.rem(i, 2)
        nxt = 1 - par
        row0 = (s + i * NS) * KB

        @pl.when(i > 0)
        def _drain():
            pltpu.make_async_copy(tab.at[pl.ds(0, KE)], xg, sem_w).wait()

        descs = [
            pltpu.async_copy(tab.at[idx_s2.at[par, j]],
                             xg.at[pl.ds(j * 128, 128)], sem)
            for j in range(KB)
        ]
        rown = jnp.minimum(row0 + NS * KB, maxrow)
        dp1 = pltpu.async_copy(src2.at[c, pl.ds(rown, KB)],
                               idx_s2.at[nxt], sem_i)
        dp2 = pltpu.async_copy(dstm.at[pl.ds(rown, KB)],
                               idx_d2.at[nxt], sem_i)
        for h in range(2):
            erh = (KB // 2) * 16
            pltpu.sync_copy(e2.at[c, pl.ds(row0 * 16 + h * erh, erh)],
                            eb.at[pl.ds(0, erh)])
            for j in range(h * KB // 2, (h + 1) * KB // 2):
                descs[j].wait()
            base = h * (KB // 2) * 128

            def _m(r, _):
                for j in range(8):
                    v = xg[base + r * 8 + j, :] + eb[r, pl.ds(j * 16, 16)]
                    xg[base + r * 8 + j, :] = jnp.maximum(v, 0.0)
                return 0
            lax.fori_loop(0, erh, _m, 0)

            for j in range(h * KB // 2, (h + 1) * KB // 2):
                pltpu.async_copy(xg.at[pl.ds(j * 128, 128)],
                                 aggr_sh.at[idx_d2.at[par, j]], sem_w,
                                 add=True)
        dp1.wait()
        dp2.wait()
        return 0
    lax.fori_loop(0, nit, _blk, 0)
    pltpu.make_async_copy(tab.at[pl.ds(0, KE)], xg, sem_w).wait()

    @pl.when(s == NS - 1)
    def _tail():
        pltpu.sync_copy(src2.at[c, pl.ds(NBLK * KB, TROW)],
                        idx_s2.at[0, pl.ds(0, TROW)])
        pltpu.sync_copy(dstm.at[pl.ds(NBLK * KB, TROW)],
                        idx_d2.at[0, pl.ds(0, TROW)])
        tdescs = [
            pltpu.async_copy(tab.at[idx_s2.at[0, j]],
                             xg.at[pl.ds(j * 128, 128)], sem)
            for j in range(TROW)
        ]
        pltpu.sync_copy(e2.at[c, pl.ds(NBLK * KB * 16, TROW * 16)],
                        eb.at[pl.ds(0, TROW * 16)])
        for d in tdescs:
            d.wait()

        def _mt(r, _):
            for j in range(8):
                v = xg[r * 8 + j, :] + eb[r, pl.ds(j * 16, 16)]
                xg[r * 8 + j, :] = jnp.maximum(v, 0.0)
            return 0
        lax.fori_loop(0, TROW * 16, _mt, 0)
        for j in range(TROW):
            pltpu.sync_copy(xg.at[pl.ds(j * 128, 128)],
                            aggr_sh.at[idx_d2.at[0, j]], add=True)

    plsc.subcore_barrier()

    pltpu.sync_copy(aggr_sh.at[pl.ds(s * NSTRIPE, NSTRIPE)],
                    out.at[c, pl.ds(s * NSTRIPE, NSTRIPE)])


def _sc_aggregate(tab, src2, dstm, e2):
    mesh = plsc.VectorSubcoreMesh(core_axis_name="c", subcore_axis_name="s",
                                  num_cores=NC, num_subcores=NS)
    f = pl.kernel(
        _sc_body,
        out_type=jax.ShapeDtypeStruct((NC, N, L), jnp.float32),
        mesh=mesh,
        compiler_params=pltpu.CompilerParams(use_tc_tiling_on_sc=False),
        scratch_types=[
            pltpu.VMEM((2, KB, 128), jnp.int32),
            pltpu.VMEM((2, KB, 128), jnp.int32),
            pltpu.VMEM((KE, L), jnp.float32),
            pltpu.VMEM((KE * 16 // 256, 128), jnp.float32),
            pltpu.VMEM_SHARED((N, L), jnp.float32),
            pltpu.SemaphoreType.DMA,
            pltpu.SemaphoreType.DMA,
            pltpu.SemaphoreType.DMA,
        ],
    )
    return f(tab, src2, dstm, e2)


BN = 4000
NG = N // BN


def _node_body(x_ref, ag_ref, bt_ref, w1_ref, b1_ref, w2_ref, b2_ref,
               gm_ref, bt2_ref, wp_ref, bp_ref, wm_ref, bm_ref,
               out_ref, s_acc, c_acc, st_acc):
    i = pl.program_id(0)

    @pl.when(i == 0)
    def _init():
        s_acc[...] = jnp.zeros((G, HID), jnp.float32)
        c_acc[...] = jnp.zeros((G, HID), jnp.float32)
        st_acc[...] = jnp.zeros((8, HID), jnp.float32)

    xb = x_ref[...]
    h0 = 2.0 * xb + jnp.concatenate(
        [ag_ref[0, :, 0:14], ag_ref[1, :, 0:14]], axis=1)
    h1 = jnp.maximum(
        jnp.dot(h0, w1_ref[...], preferred_element_type=jnp.float32)
        + b1_ref[...], 0.0)
    h2 = (jnp.dot(h1, w2_ref[...], preferred_element_type=jnp.float32)
          + b2_ref[...])
    h3 = jnp.maximum(h2, 0.0)

    bb = bt_ref[0, 0, :]
    oh = (bb[:, None] == lax.broadcasted_iota(jnp.int32, (1, G), 1)
          ).astype(jnp.float32)
    dn = (((0,), (0,)), ((), ()))
    s_acc[...] += lax.dot_general(oh, h3, dn,
                                  preferred_element_type=jnp.float32)
    c_acc[...] += lax.dot_general(oh, jnp.ones((BN, HID), jnp.float32), dn,
                                  preferred_element_type=jnp.float32)
    st_acc[0:1, :] += jnp.sum(h3, axis=0, keepdims=True)
    st_acc[1:2, :] += jnp.sum(h3 * h3, axis=0, keepdims=True)

    @pl.when(i == NG - 1)
    def _fin():
        nf = jnp.float32(N)
        mean = st_acc[0:1, :] / nf
        var = st_acc[1:2, :] / nf - mean * mean
        sf = gm_ref[...] * lax.rsqrt(var + 1e-5)
        tf = bt2_ref[...] - mean * sf
        g = s_acc[...] * sf + c_acc[...] * tf
        q = (jnp.dot(g, wp_ref[...], preferred_element_type=jnp.float32)
             + bp_ref[...])
        q = jnp.where(q > 0, q, 0.01 * q)
        o = (jnp.dot(q, wm_ref[...], preferred_element_type=jnp.float32)
             + bm_ref[...])
        out_ref[...] = 1.0 / (1.0 + jnp.exp(-o))


def _node_pipeline(x, aggr2, batch3, w1, b1, w2, b2, gm, bt, wp, bp, wm, bm):
    return pl.pallas_call(
        _node_body,
        grid=(NG,),
        in_specs=[
            pl.BlockSpec((BN, IN), lambda i: (i, 0)),
            pl.BlockSpec((2, BN, L), lambda i: (0, i, 0)),
            pl.BlockSpec((1, 1, BN), lambda i: (i, 0, 0)),
            pl.BlockSpec((IN, HID), lambda i: (0, 0)),
            pl.BlockSpec((1, HID), lambda i: (0, 0)),
            pl.BlockSpec((HID, HID), lambda i: (0, 0)),
            pl.BlockSpec((1, HID), lambda i: (0, 0)),
            pl.BlockSpec((1, HID), lambda i: (0, 0)),
            pl.BlockSpec((1, HID), lambda i: (0, 0)),
            pl.BlockSpec((HID, HID), lambda i: (0, 0)),
            pl.BlockSpec((1, HID), lambda i: (0, 0)),
            pl.BlockSpec((HID, HID), lambda i: (0, 0)),
            pl.BlockSpec((1, HID), lambda i: (0, 0)),
        ],
        out_specs=pl.BlockSpec((G, HID), lambda i: (0, 0)),
        out_shape=jax.ShapeDtypeStruct((G, HID), jnp.float32),
        scratch_shapes=[
            pltpu.VMEM((G, HID), jnp.float32),
            pltpu.VMEM((G, HID), jnp.float32),
            pltpu.VMEM((8, HID), jnp.float32),
        ],
    )(x, aggr2, batch3, w1, b1, w2, b2, gm, bt, wp, bp, wm, bm)


def kernel(x, edge_index, edge_x, batch, W_edge, b_edge, W1, b1, W2, b2,
           gamma, beta, Wp, bp, Wm, bm):
    src = edge_index[0].astype(jnp.int32)
    dst = edge_index[1].astype(jnp.int32)
    src2 = jnp.stack([src, src + NP]).reshape(NC, E // 128, 128)
    dstm = dst.reshape(E // 128, 128)
    ex_r8 = edge_x.reshape(E // 8, 8 * EDIM)
    w30 = jnp.zeros((EDIM, 30), jnp.float32).at[:, :IN].set(W_edge)
    b30 = jnp.zeros((30,), jnp.float32).at[:IN].set(b_edge)
    wb = jnp.zeros((8 * EDIM, 256), jnp.float32)
    bb = jnp.zeros((1, 256), jnp.float32)
    for a in range(8):
        for cc in range(NC):
            col = 128 * cc + 16 * a
            wb = wb.at[20 * a:20 * a + 20, col:col + 16].set(
                w30[:, 14 * cc:14 * cc + 16])
            bb = bb.at[0, col:col + 16].set(b30[14 * cc:14 * cc + 16])
    batch3 = batch.astype(jnp.int32).reshape(NG, 1, BN)

    tab_lo = jnp.pad(x[:, 0:16], ((0, NP - N), (0, 0)))
    tab_hi = jnp.pad(x[:, 14:28], ((0, NP - N), (0, 2)))
    tab128 = jnp.concatenate([tab_lo.reshape(NP // 8, 128),
                              tab_hi.reshape(NP // 8, 128)], axis=0)
    tab128 = lax.optimization_barrier(tab128)
    tab = tab128.reshape(NC * NP, L)

    e2 = _edge_mlp(ex_r8, wb, bb)
    aggr2 = _sc_aggregate(tab, src2, dstm, e2)
    return _node_pipeline(
        x, aggr2, batch3, W1, b1.reshape(1, HID), W2, b2.reshape(1, HID),
        gamma.reshape(1, HID), beta.reshape(1, HID), Wp, bp.reshape(1, HID),
        Wm, bm.reshape(1, HID))


# speedup vs baseline: 1.0680x; 1.0680x over previous
import jax
import jax.numpy as jnp
from jax import lax
from jax.experimental import pallas as pl
from jax.experimental.pallas import tpu as pltpu
from jax.experimental.pallas import tpu_sc as plsc

N = 100000
E = 1600000
IN = 28
HID = 128
G = 128
EDIM = 20

NC, NS, L = 2, 16, 16
KB = 8
KE = KB * 128
NBLK = (E // 128) // KB
TROW = (E // 128) - NBLK * KB
NSTRIPE = N // NS
NP = 102400

BR = 2000
EG = (E // 8) // BR


def _edge_body(ex_ref, w_ref, b_ref, out_ref):
    v = (jnp.dot(ex_ref[...], w_ref[...], preferred_element_type=jnp.float32)
         + b_ref[...])
    out_ref[0, :, :] = v[:, 0:128]
    out_ref[1, :, :] = v[:, 128:256]


def _edge_mlp(ex_r8, wb, bb):
    return pl.pallas_call(
        _edge_body,
        grid=(EG,),
        in_specs=[
            pl.BlockSpec((BR, 8 * EDIM), lambda i: (i, 0)),
            pl.BlockSpec((8 * EDIM, 256), lambda i: (0, 0)),
            pl.BlockSpec((1, 256), lambda i: (0, 0)),
        ],
        out_specs=pl.BlockSpec((2, BR, 128), lambda i: (0, i, 0)),
        out_shape=jax.ShapeDtypeStruct((2, E // 8, 128), jnp.float32),
    )(ex_r8, wb, bb)


def _sc_body(tab, src2, dstm, e2, out,
             idx_s2, idx_d2, xg, eb, aggr_sh, sem, sem_i, sem_w):
    c = lax.axis_index("c")
    s = lax.axis_index("s")

    def _z(i, _):
        xg[i, :] = jnp.zeros((L,), jnp.float32)
        return 0
    lax.fori_loop(0, KE, _z, 0)
    for k in range(NSTRIPE // KE):
        pltpu.sync_copy(xg.at[pl.ds(0, KE)],
                        aggr_sh.at[pl.ds(s * NSTRIPE + k * KE, KE)])
    rem = NSTRIPE - (NSTRIPE // KE) * KE
    pltpu.sync_copy(xg.at[pl.ds(0, rem)],
                    aggr_sh.at[pl.ds(s * NSTRIPE + NSTRIPE - rem, rem)])

    plsc.subcore_barrier()

    nit = (NBLK - s + NS - 1) // NS
    maxrow = (E // 128) - KB

    pltpu.sync_copy(src2.at[c, pl.ds(s * KB, KB)], idx_s2.at[0])
    pltpu.sync_copy(dstm.at[pl.ds(s * KB, KB)], idx_d2.at[0])

    def _blk(i, _):
        par = lax.rem(i, 2)
        nxt = 1 - par
        row0 = (s + i * NS) * KB

        @pl.when(i > 0)
        def _drain():
            pltpu.make_async_copy(tab.at[pl.ds(0, KE)], xg, sem_w).wait()

        descs = [
            pltpu.async_copy(tab.at[idx_s2.at[par, j]],
                             xg.at[pl.ds(j * 128, 128)], sem)
            for j in range(KB)
        ]
        rown = jnp.minimum(row0 + NS * KB, maxrow)
        dp1 = pltpu.async_copy(src2.at[c, pl.ds(rown, KB)],
                               idx_s2.at[nxt], sem_i)
        dp2 = pltpu.async_copy(dstm.at[pl.ds(rown, KB)],
                               idx_d2.at[nxt], sem_i)
        for h in range(2):
            erh = (KB // 2) * 16
            pltpu.sync_copy(e2.at[c, pl.ds(row0 * 16 + h * erh, erh)],
                            eb.at[pl.ds(0, erh)])
            for j in range(h * KB // 2, (h + 1) * KB // 2):
                descs[j].wait()
            base = h * (KB // 2) * 128

            def _m(r, _):
                for j in range(8):
                    v = xg[base + r * 8 + j, :] + eb[r, pl.ds(j * 16, 16)]
                    xg[base + r * 8 + j, :] = jnp.maximum(v, 0.0)
                return 0
            lax.fori_loop(0, erh, _m, 0)

            for j in range(h * KB // 2, (h + 1) * KB // 2):
                pltpu.async_copy(xg.at[pl.ds(j * 128, 128)],
                                 aggr_sh.at[idx_d2.at[par, j]], sem_w,
                                 add=True)
        dp1.wait()
        dp2.wait()
        return 0
    lax.fori_loop(0, nit, _blk, 0)
    pltpu.make_async_copy(tab.at[pl.ds(0, KE)], xg, sem_w).wait()

    @pl.when(s == NS - 1)
    def _tail():
        pltpu.sync_copy(src2.at[c, pl.ds(NBLK * KB, TROW)],
                        idx_s2.at[0, pl.ds(0, TROW)])
        pltpu.sync_copy(dstm.at[pl.ds(NBLK * KB, TROW)],
                        idx_d2.at[0, pl.ds(0, TROW)])
        tdescs = [
            pltpu.async_copy(tab.at[idx_s2.at[0, j]],
                             xg.at[pl.ds(j * 128, 128)], sem)
            for j in range(TROW)
        ]
        pltpu.sync_copy(e2.at[c, pl.ds(NBLK * KB * 16, TROW * 16)],
                        eb.at[pl.ds(0, TROW * 16)])
        for d in tdescs:
            d.wait()

        def _mt(r, _):
            for j in range(8):
                v = xg[r * 8 + j, :] + eb[r, pl.ds(j * 16, 16)]
                xg[r * 8 + j, :] = jnp.maximum(v, 0.0)
            return 0
        lax.fori_loop(0, TROW * 16, _mt, 0)
        for j in range(TROW):
            pltpu.sync_copy(xg.at[pl.ds(j * 128, 128)],
                            aggr_sh.at[idx_d2.at[0, j]], add=True)

    plsc.subcore_barrier()

    pltpu.sync_copy(aggr_sh.at[pl.ds(s * NSTRIPE, NSTRIPE)],
                    out.at[c, pl.ds(s * NSTRIPE, NSTRIPE)])


def _sc_aggregate(tab, src2, dstm, e2):
    mesh = plsc.VectorSubcoreMesh(core_axis_name="c", subcore_axis_name="s",
                                  num_cores=NC, num_subcores=NS)
    f = pl.kernel(
        _sc_body,
        out_type=jax.ShapeDtypeStruct((NC, N, L), jnp.float32),
        mesh=mesh,
        compiler_params=pltpu.CompilerParams(use_tc_tiling_on_sc=False),
        scratch_types=[
            pltpu.VMEM((2, KB, 128), jnp.int32),
            pltpu.VMEM((2, KB, 128), jnp.int32),
            pltpu.VMEM((KE, L), jnp.float32),
            pltpu.VMEM((KE * 16 // 256, 128), jnp.float32),
            pltpu.VMEM_SHARED((N, L), jnp.float32),
            pltpu.SemaphoreType.DMA,
            pltpu.SemaphoreType.DMA,
            pltpu.SemaphoreType.DMA,
        ],
    )
    return f(tab, src2, dstm, e2)


BN = 4000
NG = N // BN


def _node_body(x_ref, ag_ref, bt_ref, w1_ref, b1_ref, w2_ref, b2_ref,
               gm_ref, bt2_ref, wp_ref, bp_ref, wm_ref, bm_ref,
               out_ref, s_acc, c_acc, st_acc):
    i = pl.program_id(0)

    @pl.when(i == 0)
    def _init():
        s_acc[...] = jnp.zeros((G, HID), jnp.float32)
        c_acc[...] = jnp.zeros((G, HID), jnp.float32)
        st_acc[...] = jnp.zeros((8, HID), jnp.float32)

    xb = x_ref[...]
    h0 = 2.0 * xb + jnp.concatenate(
        [ag_ref[0, :, 0:14], ag_ref[1, :, 0:14]], axis=1)
    h1 = jnp.maximum(
        jnp.dot(h0, w1_ref[...], preferred_element_type=jnp.float32)
        + b1_ref[...], 0.0)
    h2 = (jnp.dot(h1, w2_ref[...], preferred_element_type=jnp.float32)
          + b2_ref[...])
    h3 = jnp.maximum(h2, 0.0)

    bb = bt_ref[0, 0, :]
    oh = (bb[:, None] == lax.broadcasted_iota(jnp.int32, (1, G), 1)
          ).astype(jnp.float32)
    dn = (((0,), (0,)), ((), ()))
    s_acc[...] += lax.dot_general(oh, h3, dn,
                                  preferred_element_type=jnp.float32)
    c_acc[...] += lax.dot_general(oh, jnp.ones((BN, HID), jnp.float32), dn,
                                  preferred_element_type=jnp.float32)
    st_acc[0:1, :] += jnp.sum(h3, axis=0, keepdims=True)
    st_acc[1:2, :] += jnp.sum(h3 * h3, axis=0, keepdims=True)

    @pl.when(i == NG - 1)
    def _fin():
        nf = jnp.float32(N)
        mean = st_acc[0:1, :] / nf
        var = st_acc[1:2, :] / nf - mean * mean
        sf = gm_ref[...] * lax.rsqrt(var + 1e-5)
        tf = bt2_ref[...] - mean * sf
        g = s_acc[...] * sf + c_acc[...] * tf
        q = (jnp.dot(g, wp_ref[...], preferred_element_type=jnp.float32)
             + bp_ref[...])
        q = jnp.where(q > 0, q, 0.01 * q)
        o = (jnp.dot(q, wm_ref[...], preferred_element_type=jnp.float32)
             + bm_ref[...])
        out_ref[...] = 1.0 / (1.0 + jnp.exp(-o))


def _node_pipeline(x, aggr2, batch3, w1, b1, w2, b2, gm, bt, wp, bp, wm, bm):
    return pl.pallas_call(
        _node_body,
        grid=(NG,),
        in_specs=[
            pl.BlockSpec((BN, IN), lambda i: (i, 0)),
            pl.BlockSpec((2, BN, L), lambda i: (0, i, 0)),
            pl.BlockSpec((1, 1, BN), lambda i: (i, 0, 0)),
            pl.BlockSpec((IN, HID), lambda i: (0, 0)),
            pl.BlockSpec((1, HID), lambda i: (0, 0)),
            pl.BlockSpec((HID, HID), lambda i: (0, 0)),
            pl.BlockSpec((1, HID), lambda i: (0, 0)),
            pl.BlockSpec((1, HID), lambda i: (0, 0)),
            pl.BlockSpec((1, HID), lambda i: (0, 0)),
            pl.BlockSpec((HID, HID), lambda i: (0, 0)),
            pl.BlockSpec((1, HID), lambda i: (0, 0)),
            pl.BlockSpec((HID, HID), lambda i: (0, 0)),
            pl.BlockSpec((1, HID), lambda i: (0, 0)),
        ],
        out_specs=pl.BlockSpec((G, HID), lambda i: (0, 0)),
        out_shape=jax.ShapeDtypeStruct((G, HID), jnp.float32),
        scratch_shapes=[
            pltpu.VMEM((G, HID), jnp.float32),
            pltpu.VMEM((G, HID), jnp.float32),
            pltpu.VMEM((8, HID), jnp.float32),
        ],
    )(x, aggr2, batch3, w1, b1, w2, b2, gm, bt, wp, bp, wm, bm)


def kernel(x, edge_index, edge_x, batch, W_edge, b_edge, W1, b1, W2, b2,
           gamma, beta, Wp, bp, Wm, bm):
    src = edge_index[0].astype(jnp.int32)
    dst = edge_index[1].astype(jnp.int32)
    src2 = jnp.stack([src, src + NP]).reshape(NC, E // 128, 128)
    dstm = dst.reshape(E // 128, 128)
    ex_r8 = edge_x.reshape(E // 8, 8 * EDIM)
    w30 = jnp.zeros((EDIM, 30), jnp.float32).at[:, :IN].set(W_edge)
    b30 = jnp.zeros((30,), jnp.float32).at[:IN].set(b_edge)
    wb = jnp.zeros((8 * EDIM, 256), jnp.float32)
    bb = jnp.zeros((1, 256), jnp.float32)
    for a in range(8):
        for cc in range(NC):
            col = 128 * cc + 16 * a
            wb = wb.at[20 * a:20 * a + 20, col:col + 16].set(
                w30[:, 14 * cc:14 * cc + 16])
            bb = bb.at[0, col:col + 16].set(b30[14 * cc:14 * cc + 16])
    batch3 = batch.astype(jnp.int32).reshape(NG, 1, BN)

    tab = jnp.concatenate(
        [jnp.pad(x[:, 0:16], ((0, NP - N), (0, 0))),
         jnp.pad(x[:, 14:28], ((0, NP - N), (0, 2)))], axis=0)

    e2 = _edge_mlp(ex_r8, wb, bb)
    aggr2 = _sc_aggregate(tab, src2, dstm, e2)
    return _node_pipeline(
        x, aggr2, batch3, W1, b1.reshape(1, HID), W2, b2.reshape(1, HID),
        gamma.reshape(1, HID), beta.reshape(1, HID), Wp, bp.reshape(1, HID),
        Wm, bm.reshape(1, HID))
